# baseline (device time: 42227 ns/iter reference)
import jax
import jax.numpy as jnp
from jax import lax
from jax.experimental import pallas as pl
from jax.experimental.pallas import tpu as pltpu

K = 64
CH = 64
NC = 9
NY = 7


def kernel(partial, resid, gamma):
    _, m, d = partial.shape
    own = m // 2 + K
    share = m // 2 - K
    assert own == NC * CH and share == NY * CH

    def body(p_ref, r_ref, g_ref, o_ref, pbuf, rbuf, xcomm, obuf, ycomm,
             p_sem, r_sem, store_sems, ystore_sems, x_send_sems,
             x_recv_sems, y_send_sems, y_recv_sems):
        my_x = lax.axis_index("x")
        my_y = lax.axis_index("y")
        x_peer = (1 - my_x, my_y)
        y_peer = (my_x, 1 - my_y)
        base = my_y * share

        my_rows = pl.ds(base, own)
        pcopy = pltpu.make_async_copy(p_ref.at[0, my_rows, :], pbuf, p_sem)
        pcopy.start()
        rcopy = pltpu.make_async_copy(r_ref.at[my_rows, :], rbuf, r_sem)
        rcopy.start()

        barrier_sem = pltpu.get_barrier_semaphore()
        for peer in (x_peer, y_peer):
            pl.semaphore_signal(
                barrier_sem, inc=1, device_id=peer,
                device_id_type=pl.DeviceIdType.MESH,
            )
        pl.semaphore_wait(barrier_sem, 2)
        pcopy.wait()

        x_rdmas = []
        for c in range(NC):
            lrows = pl.ds(c * CH, CH)
            rdma = pltpu.make_async_remote_copy(
                src_ref=pbuf.at[lrows, :],
                dst_ref=xcomm.at[lrows, :],
                send_sem=x_send_sems.at[c],
                recv_sem=x_recv_sems.at[c],
                device_id=x_peer,
                device_id_type=pl.DeviceIdType.MESH,
            )
            rdma.start()
            x_rdmas.append(rdma)

        rcopy.wait()

        y_rdmas = {}
        stores = []
        for c in range(NC):
            lrows = pl.ds(c * CH, CH)
            x_rdmas[c].wait_recv()
            y = pbuf[lrows, :] + xcomm[lrows, :] + rbuf[lrows, :]
            rms = jnp.sqrt(jnp.mean(y * y, axis=-1, keepdims=True) + 1e-6)
            obuf[lrows, :] = y / rms * g_ref[...][None, :]
            st = pltpu.make_async_copy(
                obuf.at[lrows, :], o_ref.at[pl.ds(base + c * CH, CH), :],
                store_sems.at[c],
            )
            st.start()
            stores.append(st)

            def mk_ysend(c=c):
                return pltpu.make_async_remote_copy(
                    src_ref=obuf.at[pl.ds(c * CH, CH), :],
                    dst_ref=ycomm.at[pl.ds(c * CH - my_y * 2 * CH, CH), :],
                    send_sem=y_send_sems.at[c],
                    recv_sem=y_recv_sems.at[c],
                    device_id=y_peer,
                    device_id_type=pl.DeviceIdType.MESH,
                )

            y_rdmas[c] = mk_ysend()
            if c < NC - NY:
                @pl.when(my_y == 0)
                def _(c=c):
                    mk_ysend(c).start()
            elif c >= NY:
                @pl.when(my_y == 1)
                def _(c=c):
                    mk_ysend(c).start()
            else:
                y_rdmas[c].start()

        ystores = []
        for c in range(NC):
            def drain(c=c):
                rdma = pltpu.make_async_remote_copy(
                    src_ref=obuf.at[pl.ds(c * CH, CH), :],
                    dst_ref=ycomm.at[pl.ds(c * CH - 2 * CH + my_y * 2 * CH, CH), :],
                    send_sem=y_send_sems.at[c],
                    recv_sem=y_recv_sems.at[c],
                    device_id=y_peer,
                    device_id_type=pl.DeviceIdType.MESH,
                )
                rdma.wait_recv()
                st = pltpu.make_async_copy(
                    ycomm.at[pl.ds(c * CH - 2 * CH + my_y * 2 * CH, CH), :],
                    o_ref.at[pl.ds((1 - my_y) * share + c * CH, CH), :],
                    ystore_sems.at[c],
                )
                st.start()
                return st

            if c < NC - NY:
                @pl.when(my_y == 1)
                def _(c=c):
                    drain(c).wait()
            elif c >= NY:
                @pl.when(my_y == 0)
                def _(c=c):
                    drain(c).wait()
            else:
                ystores.append(drain(c))

        for st in ystores:
            st.wait()
        for c in range(NC):
            stores[c].wait()
            x_rdmas[c].wait_send()
            if c < NC - NY:
                @pl.when(my_y == 0)
                def _(c=c):
                    y_rdmas[c].wait_send()
            elif c >= NY:
                @pl.when(my_y == 1)
                def _(c=c):
                    y_rdmas[c].wait_send()
            else:
                y_rdmas[c].wait_send()

    return pl.pallas_call(
        body,
        out_shape=jax.ShapeDtypeStruct((m, d), jnp.float32),
        in_specs=[
            pl.BlockSpec(memory_space=pl.ANY),
            pl.BlockSpec(memory_space=pl.ANY),
            pl.BlockSpec(memory_space=pltpu.VMEM),
        ],
        out_specs=pl.BlockSpec(memory_space=pl.ANY),
        scratch_shapes=[
            pltpu.VMEM((NC * CH, d), jnp.float32),
            pltpu.VMEM((NC * CH, d), jnp.float32),
            pltpu.VMEM((NC * CH, d), jnp.float32),
            pltpu.VMEM((NC * CH, d), jnp.float32),
            pltpu.VMEM((NY * CH, d), jnp.float32),
            pltpu.SemaphoreType.DMA,
            pltpu.SemaphoreType.DMA,
            pltpu.SemaphoreType.DMA((NC,)),
            pltpu.SemaphoreType.DMA((NC,)),
            pltpu.SemaphoreType.DMA((NC,)),
            pltpu.SemaphoreType.DMA((NC,)),
            pltpu.SemaphoreType.DMA((NC,)),
            pltpu.SemaphoreType.DMA((NC,)),
        ],
        compiler_params=pltpu.CompilerParams(collective_id=0),
    )(partial, resid, gamma)


# device time: 39139 ns/iter; 1.0789x vs baseline; 1.0789x over previous
import jax
import jax.numpy as jnp
from jax import lax
from jax.experimental import pallas as pl
from jax.experimental.pallas import tpu as pltpu

N_CHUNKS = 8


def kernel(partial, resid, gamma):
    _, m, d = partial.shape
    half = m // 2
    rc = half // N_CHUNKS

    def body(p_ref, r_ref, g_ref, o_ref, pbuf, rbuf, xcomm,
             p_sem, r_sem, x_send_sems, x_recv_sems,
             y_send_sems, y_recv_sems):
        my_x = lax.axis_index("x")
        my_y = lax.axis_index("y")
        x_peer = (1 - my_x, my_y)
        y_peer = (my_x, 1 - my_y)
        half_off = my_y * half

        my_rows = pl.ds(half_off, half)
        pcopy = pltpu.make_async_copy(p_ref.at[0, my_rows, :], pbuf, p_sem)
        pcopy.start()
        rcopy = pltpu.make_async_copy(r_ref.at[my_rows, :], rbuf, r_sem)
        rcopy.start()

        barrier_sem = pltpu.get_barrier_semaphore()
        for peer in (x_peer, y_peer):
            pl.semaphore_signal(
                barrier_sem, inc=1, device_id=peer,
                device_id_type=pl.DeviceIdType.MESH,
            )
        pl.semaphore_wait(barrier_sem, 2)
        pcopy.wait()

        x_rdmas = []
        for c in range(N_CHUNKS):
            lrows = pl.ds(c * rc, rc)
            rdma = pltpu.make_async_remote_copy(
                src_ref=pbuf.at[lrows, :],
                dst_ref=xcomm.at[lrows, :],
                send_sem=x_send_sems.at[c],
                recv_sem=x_recv_sems.at[c],
                device_id=x_peer,
                device_id_type=pl.DeviceIdType.MESH,
            )
            rdma.start()
            x_rdmas.append(rdma)

        rcopy.wait()

        y_rdmas = []
        for c in range(N_CHUNKS):
            lrows = pl.ds(c * rc, rc)
            rows = pl.ds(half_off + c * rc, rc)
            x_rdmas[c].wait_recv()
            y = pbuf[lrows, :] + xcomm[lrows, :] + rbuf[lrows, :]
            rms = jnp.sqrt(jnp.mean(y * y, axis=-1, keepdims=True) + 1e-6)
            o_ref[rows, :] = y / rms * g_ref[...][None, :]
            rdma = pltpu.make_async_remote_copy(
                src_ref=o_ref.at[rows, :],
                dst_ref=o_ref.at[rows, :],
                send_sem=y_send_sems.at[c],
                recv_sem=y_recv_sems.at[c],
                device_id=y_peer,
                device_id_type=pl.DeviceIdType.MESH,
            )
            rdma.start()
            y_rdmas.append(rdma)

        for c in range(N_CHUNKS):
            x_rdmas[c].wait_send()
            y_rdmas[c].wait_send()
            y_rdmas[c].wait_recv()

    return pl.pallas_call(
        body,
        out_shape=jax.ShapeDtypeStruct((m, d), jnp.float32),
        in_specs=[
            pl.BlockSpec(memory_space=pl.ANY),
            pl.BlockSpec(memory_space=pl.ANY),
            pl.BlockSpec(memory_space=pltpu.VMEM),
        ],
        out_specs=pl.BlockSpec(memory_space=pltpu.VMEM),
        scratch_shapes=[
            pltpu.VMEM((half, d), jnp.float32),
            pltpu.VMEM((half, d), jnp.float32),
            pltpu.VMEM((half, d), jnp.float32),
            pltpu.SemaphoreType.DMA,
            pltpu.SemaphoreType.DMA,
            pltpu.SemaphoreType.DMA((N_CHUNKS,)),
            pltpu.SemaphoreType.DMA((N_CHUNKS,)),
            pltpu.SemaphoreType.DMA((N_CHUNKS,)),
            pltpu.SemaphoreType.DMA((N_CHUNKS,)),
        ],
        compiler_params=pltpu.CompilerParams(collective_id=0),
    )(partial, resid, gamma)
